# Initial kernel scaffold; baseline (speedup 1.0000x reference)
import functools

import jax
import jax.numpy as jnp
from jax import lax
from jax.experimental import pallas as pl
from jax.experimental.pallas import tpu as pltpu
from jax.experimental.pallas import tpu_sc as plsc

F32 = jnp.float32


def _cdiv(a, b):
    return (a + b - 1) // b


def _build_gather(N, D, En, NW, mesh):
    CG = 512
    G = En // CG
    KMAX = _cdiv(G, NW)
    NC = 2

    @functools.partial(
        pl.kernel,
        out_type=(
            jax.ShapeDtypeStruct((En, D), F32),
            jax.ShapeDtypeStruct((En, D), F32),
        ),
        mesh=mesh,
        scratch_types=[
            pltpu.VMEM((CG,), jnp.int32),
            pltpu.VMEM((CG, D), F32),
            pltpu.SemaphoreType.DMA,
        ],
    )
    def gather_k(v_hbm, i0_hbm, i1_hbm, s_out, r_out, idx_v, rows_v, gsem):
        cid = lax.axis_index("c")
        sid = lax.axis_index("s")
        wid = sid * NC + cid

        def one_array(i_hbm, out_hbm):
            def body(k, carry):
                g = k * NW + wid

                @pl.when(g < G)
                def _():
                    base = pl.multiple_of(g * CG, CG)
                    pltpu.sync_copy(i_hbm.at[pl.ds(base, CG)], idx_v)
                    descs = [
                        pltpu.async_copy(
                            v_hbm.at[idx_v.at[pl.ds(j * 128, 128)]],
                            rows_v.at[pl.ds(j * 128, 128)],
                            gsem,
                        )
                        for j in range(CG // 128)
                    ]
                    for d in descs:
                        d.wait()
                    pltpu.sync_copy(rows_v, out_hbm.at[pl.ds(base, CG)])

                return carry

            lax.fori_loop(0, KMAX, body, 0)

        one_array(i0_hbm, s_out)
        one_array(i1_hbm, r_out)

    return gather_k


def _build_scatter(NPAD, En, NW, mesh):
    H = 64
    G = En // 128
    KMAX = _cdiv(G, NW)
    NC = 2
    RPT = NPAD // 16

    @functools.partial(
        pl.kernel,
        out_type=(
            jax.ShapeDtypeStruct((2 * NPAD, H), F32),
            jax.ShapeDtypeStruct((2 * NPAD, H), F32),
            jax.ShapeDtypeStruct((NW, NPAD), F32),
            jax.ShapeDtypeStruct((NW, NPAD), F32),
        ),
        mesh=mesh,
        scratch_types=[
            pltpu.VMEM_SHARED((NPAD, H), F32),
            pltpu.VMEM_SHARED((NPAD, H), F32),
            pltpu.VMEM((128,), jnp.int32),
            pltpu.VMEM((128,), jnp.int32),
            pltpu.VMEM((128, H), F32),
            pltpu.VMEM((128, H), F32),
            pltpu.VMEM((NPAD,), F32),
            pltpu.VMEM((NPAD,), F32),
            pltpu.VMEM((RPT, H), F32),
        ],
    )
    def scatter_k(ee_hbm, i0_hbm, i1_hbm, z64_hbm, zc_hbm,
                  s0_out, s1_out, c0_out, c1_out,
                  s0_sh, s1_sh, i0_v, i1_v, e0_v, e1_v, c0_v, c1_v, wr_v):
        cid = lax.axis_index("c")
        sid = lax.axis_index("s")
        wid = sid * NC + cid
        ones16 = jnp.full((16,), 1.0, dtype=F32)

        pltpu.sync_copy(z64_hbm, s0_sh.at[pl.ds(sid * RPT, RPT)])
        pltpu.sync_copy(z64_hbm, s1_sh.at[pl.ds(sid * RPT, RPT)])
        pltpu.sync_copy(zc_hbm, c0_v)
        pltpu.sync_copy(zc_hbm, c1_v)
        plsc.subcore_barrier()

        def body(k, carry):
            g = k * NW + wid

            @pl.when(g < G)
            def _():
                base = pl.multiple_of(g * 128, 128)
                pltpu.sync_copy(i0_hbm.at[pl.ds(base, 128)], i0_v)
                pltpu.sync_copy(i1_hbm.at[pl.ds(base, 128)], i1_v)
                pltpu.sync_copy(ee_hbm.at[pl.ds(base, 128), pl.ds(0, H)], e0_v)
                pltpu.sync_copy(ee_hbm.at[pl.ds(base, 128), pl.ds(H, H)], e1_v)
                pltpu.sync_copy(e0_v, s0_sh.at[i0_v], add=True)
                pltpu.sync_copy(e1_v, s1_sh.at[i1_v], add=True)
                for j in range(8):
                    plsc.addupdate_scatter(
                        c0_v, [i0_v[pl.ds(j * 16, 16)]], ones16)
                    plsc.addupdate_scatter(
                        c1_v, [i1_v[pl.ds(j * 16, 16)]], ones16)

            return carry

        lax.fori_loop(0, KMAX, body, 0)
        plsc.subcore_barrier()

        off = cid * NPAD + sid * RPT
        pltpu.sync_copy(s0_sh.at[pl.ds(sid * RPT, RPT)], wr_v)
        pltpu.sync_copy(wr_v, s0_out.at[pl.ds(off, RPT)])
        pltpu.sync_copy(s1_sh.at[pl.ds(sid * RPT, RPT)], wr_v)
        pltpu.sync_copy(wr_v, s1_out.at[pl.ds(off, RPT)])
        pltpu.sync_copy(c0_v, c0_out.at[wid])
        pltpu.sync_copy(c1_v, c1_out.at[wid])

    return scatter_k


def _edge_mlp(senders, receivers, E2, W1e, b1e, W2e, b2e):
    En, D = E2.shape
    BE = 512
    grid = En // BE

    def body(s_ref, r_ref, e_ref, w1_ref, b1_ref, w2_ref, b2_ref, o_ref):
        x = jnp.concatenate([s_ref[...], r_ref[...], e_ref[...]], axis=1)
        h = jnp.dot(x, w1_ref[...], preferred_element_type=F32) + b1_ref[...]
        h = h * (1.0 / (1.0 + jnp.exp(-h)))
        o_ref[...] = (
            jnp.dot(h, w2_ref[...], preferred_element_type=F32) + b2_ref[...]
        )

    return pl.pallas_call(
        body,
        grid=(grid,),
        in_specs=[
            pl.BlockSpec((BE, D), lambda i: (i, 0)),
            pl.BlockSpec((BE, D), lambda i: (i, 0)),
            pl.BlockSpec((BE, D), lambda i: (i, 0)),
            pl.BlockSpec(W1e.shape, lambda i: (0, 0)),
            pl.BlockSpec((1, 128), lambda i: (0, 0)),
            pl.BlockSpec(W2e.shape, lambda i: (0, 0)),
            pl.BlockSpec((1, D), lambda i: (0, 0)),
        ],
        out_specs=pl.BlockSpec((BE, D), lambda i: (i, 0)),
        out_shape=jax.ShapeDtypeStruct((En, D), F32),
    )(senders, receivers, E2, W1e, b1e.reshape(1, -1), W2e, b2e.reshape(1, -1))


def _node_mlp(V_pad, s0f, s1f, c0p, c1p, W1n, b1n, W2n, b2n, NPAD, NW):
    D = V_pad.shape[1]
    H = 64
    BN = 1024
    grid = NPAD // BN
    s0r = s0f.reshape(2, NPAD, H)
    s1r = s1f.reshape(2, NPAD, H)

    def body(v_ref, s0_ref, s1_ref, c0_ref, c1_ref,
             w1_ref, b1_ref, w2_ref, b2_ref, o_ref):
        s0 = s0_ref[0] + s0_ref[1]
        s1 = s1_ref[0] + s1_ref[1]
        c0 = jnp.sum(c0_ref[...], axis=0)[:, None]
        c1 = jnp.sum(c1_ref[...], axis=0)[:, None]
        m0 = s0 / jnp.maximum(c0, 1.0)
        m1 = s1 / jnp.maximum(c1, 1.0)
        x = jnp.concatenate([v_ref[...], m0, m1], axis=1)
        h = jnp.dot(x, w1_ref[...], preferred_element_type=F32) + b1_ref[...]
        h = h * (1.0 / (1.0 + jnp.exp(-h)))
        o_ref[...] = (
            jnp.dot(h, w2_ref[...], preferred_element_type=F32) + b2_ref[...]
        )

    return pl.pallas_call(
        body,
        grid=(grid,),
        in_specs=[
            pl.BlockSpec((BN, D), lambda i: (i, 0)),
            pl.BlockSpec((2, BN, H), lambda i: (0, i, 0)),
            pl.BlockSpec((2, BN, H), lambda i: (0, i, 0)),
            pl.BlockSpec((NW, BN), lambda i: (0, i)),
            pl.BlockSpec((NW, BN), lambda i: (0, i)),
            pl.BlockSpec(W1n.shape, lambda i: (0, 0)),
            pl.BlockSpec((1, 128), lambda i: (0, 0)),
            pl.BlockSpec(W2n.shape, lambda i: (0, 0)),
            pl.BlockSpec((1, D), lambda i: (0, 0)),
        ],
        out_specs=pl.BlockSpec((BN, D), lambda i: (i, 0)),
        out_shape=jax.ShapeDtypeStruct((NPAD, D), F32),
    )(V_pad, s0r, s1r, c0p, c1p,
      W1n, b1n.reshape(1, -1), W2n, b2n.reshape(1, -1))


def kernel(V, E, edges, W1e, b1e, W2e, b2e, W1n, b1n, W2n, b2n):
    _, N, D = V.shape
    En = E.shape[1]
    NPAD = 10240
    info = plsc.get_sparse_core_info()
    NW = info.num_cores * info.num_subcores
    mesh = plsc.VectorSubcoreMesh(core_axis_name="c", subcore_axis_name="s")

    V2 = V[0]
    E2 = E[0]
    idx0 = edges[0, :, 0]
    idx1 = edges[0, :, 1]

    senders, receivers = _build_gather(N, D, En, NW, mesh)(V2, idx0, idx1)

    edge_emb = _edge_mlp(senders, receivers, E2, W1e, b1e, W2e, b2e)

    z64 = jnp.zeros((NPAD // 16, 64), dtype=F32)
    zc = jnp.zeros((NPAD,), dtype=F32)
    s0f, s1f, c0p, c1p = _build_scatter(NPAD, En, NW, mesh)(
        edge_emb, idx0, idx1, z64, zc)

    V_pad = jnp.pad(V2, ((0, NPAD - N), (0, 0)))
    node_pad = _node_mlp(
        V_pad, s0f, s1f, c0p, c1p, W1n, b1n, W2n, b2n, NPAD, NW)

    return node_pad[:N][None], edge_emb[None]


# speedup vs baseline: 2.6937x; 2.6937x over previous
import functools

import jax
import jax.numpy as jnp
from jax import lax
from jax.experimental import pallas as pl
from jax.experimental.pallas import tpu as pltpu
from jax.experimental.pallas import tpu_sc as plsc

F32 = jnp.float32


def _cdiv(a, b):
    return (a + b - 1) // b


def _build_gather(N, D, En, NPAD, NW, mesh):
    CG = 512
    G = En // CG
    KMAX = _cdiv(G, NW)
    NC = 2

    @functools.partial(
        pl.kernel,
        out_type=(
            jax.ShapeDtypeStruct((En, D), F32),
            jax.ShapeDtypeStruct((En, D), F32),
            jax.ShapeDtypeStruct((NW, NPAD), F32),
            jax.ShapeDtypeStruct((NW, NPAD), F32),
        ),
        mesh=mesh,
        scratch_types=[
            pltpu.VMEM((CG,), jnp.int32),
            pltpu.VMEM((CG, D), F32),
            pltpu.VMEM((NPAD,), F32),
            pltpu.VMEM((NPAD,), F32),
            pltpu.SemaphoreType.DMA,
        ],
        compiler_params=pltpu.CompilerParams(needs_layout_passes=False),
    )
    def gather_k(v_hbm, i0_hbm, i1_hbm, zc_hbm,
                 s_out, r_out, c0_out, c1_out,
                 idx_v, rows_v, c0_v, c1_v, gsem):
        cid = lax.axis_index("c")
        sid = lax.axis_index("s")
        wid = sid * NC + cid
        ones16 = jnp.full((16,), 1.0, dtype=F32)

        pltpu.sync_copy(zc_hbm, c0_v)
        pltpu.sync_copy(zc_hbm, c1_v)

        def one_array(i_hbm, out_hbm, c_v):
            def body(k, carry):
                g = k * NW + wid

                @pl.when(g < G)
                def _():
                    base = pl.multiple_of(g * CG, CG)
                    pltpu.sync_copy(i_hbm.at[pl.ds(base, CG)], idx_v)
                    descs = [
                        pltpu.async_copy(
                            v_hbm.at[idx_v.at[pl.ds(j * 128, 128)]],
                            rows_v.at[pl.ds(j * 128, 128)],
                            gsem,
                        )
                        for j in range(CG // 128)
                    ]
                    for j in range(CG // 16):
                        plsc.addupdate_scatter(
                            c_v, [idx_v[pl.ds(j * 16, 16)]], ones16)
                    for d in descs:
                        d.wait()
                    pltpu.sync_copy(rows_v, out_hbm.at[pl.ds(base, CG)])

                return carry

            lax.fori_loop(0, KMAX, body, 0)

        one_array(i0_hbm, s_out, c0_v)
        one_array(i1_hbm, r_out, c1_v)

        pltpu.sync_copy(c0_v, c0_out.at[wid])
        pltpu.sync_copy(c1_v, c1_out.at[wid])

    return gather_k


def _build_scatter(NPAD, En, NW, mesh):
    H = 64
    G = En // 128
    NS = 16
    KMAX = _cdiv(G, NS)
    RPT = NPAD // 16

    @functools.partial(
        pl.kernel,
        out_type=jax.ShapeDtypeStruct((2 * NPAD, H), F32),
        mesh=mesh,
        scratch_types=[
            pltpu.VMEM_SHARED((NPAD, H), F32),
            pltpu.VMEM((1, 128), jnp.int32),
            pltpu.VMEM((128, H), F32),
            pltpu.VMEM((RPT, H), F32),
        ],
        compiler_params=pltpu.CompilerParams(use_tc_tiling_on_sc=False),
    )
    def scatter_k(ee_hbm, idx_hbm, z64_hbm,
                  s_out,
                  s_sh, idx_v, e_v, wr_v):
        cid = lax.axis_index("c")
        sid = lax.axis_index("s")

        pltpu.sync_copy(z64_hbm, s_sh.at[pl.ds(sid * RPT, RPT)])
        plsc.subcore_barrier()

        def body(k, carry):
            g = k * NS + sid

            @pl.when(g < G)
            def _():
                pltpu.sync_copy(idx_hbm.at[pl.ds(2 * g + cid, 1)], idx_v)
                pltpu.sync_copy(
                    ee_hbm.at[pl.ds(En * cid + 128 * g, 128)], e_v)
                pltpu.sync_copy(e_v, s_sh.at[idx_v.at[0]], add=True)

            return carry

        lax.fori_loop(0, KMAX, body, 0)
        plsc.subcore_barrier()

        off = cid * NPAD + sid * RPT
        pltpu.sync_copy(s_sh.at[pl.ds(sid * RPT, RPT)], wr_v)
        pltpu.sync_copy(wr_v, s_out.at[pl.ds(off, RPT)])

    return scatter_k


def _edge_mlp(senders, receivers, E2, W1e, b1e, W2e, b2e):
    En, D = E2.shape
    BE = 512
    grid = En // BE

    def body(s_ref, r_ref, e_ref, w1_ref, b1_ref, w2_ref, b2_ref,
             o_ref, o2_ref):
        x = jnp.concatenate([s_ref[...], r_ref[...], e_ref[...]], axis=1)
        h = jnp.dot(x, w1_ref[...], preferred_element_type=F32) + b1_ref[...]
        h = h * (1.0 / (1.0 + jnp.exp(-h)))
        out = (
            jnp.dot(h, w2_ref[...], preferred_element_type=F32) + b2_ref[...]
        )
        o_ref[...] = out
        o2_ref[0] = out[:, :64]
        o2_ref[1] = out[:, 64:]

    return pl.pallas_call(
        body,
        grid=(grid,),
        in_specs=[
            pl.BlockSpec((BE, D), lambda i: (i, 0)),
            pl.BlockSpec((BE, D), lambda i: (i, 0)),
            pl.BlockSpec((BE, D), lambda i: (i, 0)),
            pl.BlockSpec(W1e.shape, lambda i: (0, 0)),
            pl.BlockSpec((1, 128), lambda i: (0, 0)),
            pl.BlockSpec(W2e.shape, lambda i: (0, 0)),
            pl.BlockSpec((1, D), lambda i: (0, 0)),
        ],
        out_specs=[
            pl.BlockSpec((BE, D), lambda i: (i, 0)),
            pl.BlockSpec((2, BE, 64), lambda i: (0, i, 0)),
        ],
        out_shape=[
            jax.ShapeDtypeStruct((En, D), F32),
            jax.ShapeDtypeStruct((2, En, 64), F32),
        ],
    )(senders, receivers, E2, W1e, b1e.reshape(1, -1), W2e, b2e.reshape(1, -1))


def _node_mlp(V_pad, sf, c0p, c1p, W1n, b1n, W2n, b2n, NPAD, NW):
    D = V_pad.shape[1]
    H = 64
    BN = 1024
    grid = NPAD // BN
    sr = sf.reshape(2, NPAD, H)

    def body(v_ref, s_ref, c0_ref, c1_ref,
             w1_ref, b1_ref, w2_ref, b2_ref, o_ref):
        s0 = s_ref[0]
        s1 = s_ref[1]
        c0 = jnp.sum(c0_ref[...], axis=0)[:, None]
        c1 = jnp.sum(c1_ref[...], axis=0)[:, None]
        m0 = s0 / jnp.maximum(c0, 1.0)
        m1 = s1 / jnp.maximum(c1, 1.0)
        x = jnp.concatenate([v_ref[...], m0, m1], axis=1)
        h = jnp.dot(x, w1_ref[...], preferred_element_type=F32) + b1_ref[...]
        h = h * (1.0 / (1.0 + jnp.exp(-h)))
        o_ref[...] = (
            jnp.dot(h, w2_ref[...], preferred_element_type=F32) + b2_ref[...]
        )

    return pl.pallas_call(
        body,
        grid=(grid,),
        in_specs=[
            pl.BlockSpec((BN, D), lambda i: (i, 0)),
            pl.BlockSpec((2, BN, H), lambda i: (0, i, 0)),
            pl.BlockSpec((NW, BN), lambda i: (0, i)),
            pl.BlockSpec((NW, BN), lambda i: (0, i)),
            pl.BlockSpec(W1n.shape, lambda i: (0, 0)),
            pl.BlockSpec((1, 128), lambda i: (0, 0)),
            pl.BlockSpec(W2n.shape, lambda i: (0, 0)),
            pl.BlockSpec((1, D), lambda i: (0, 0)),
        ],
        out_specs=pl.BlockSpec((BN, D), lambda i: (i, 0)),
        out_shape=jax.ShapeDtypeStruct((NPAD, D), F32),
    )(V_pad, sr, c0p, c1p,
      W1n, b1n.reshape(1, -1), W2n, b2n.reshape(1, -1))


def kernel(V, E, edges, W1e, b1e, W2e, b2e, W1n, b1n, W2n, b2n):
    _, N, D = V.shape
    En = E.shape[1]
    NPAD = 10240
    info = plsc.get_sparse_core_info()
    NW = info.num_cores * info.num_subcores
    mesh = plsc.VectorSubcoreMesh(core_axis_name="c", subcore_axis_name="s")

    V2 = V[0]
    E2 = E[0]
    idx0 = edges[0, :, 0]
    idx1 = edges[0, :, 1]
    idx_comb = jnp.stack(
        [idx0.reshape(-1, 128), idx1.reshape(-1, 128)], axis=1
    ).reshape(-1, 128)

    RPT = NPAD // 16
    zc = jnp.zeros((NPAD,), dtype=F32)
    senders, receivers, c0p, c1p = _build_gather(N, D, En, NPAD, NW, mesh)(
        V2, idx0, idx1, zc)

    edge_emb, ee_half = _edge_mlp(senders, receivers, E2, W1e, b1e, W2e, b2e)

    ee_r = ee_half.reshape(2 * En, 64)
    z64 = jnp.zeros((RPT, 64), dtype=F32)
    sf = _build_scatter(NPAD, En, NW, mesh)(ee_r, idx_comb, z64)

    V_pad = jnp.pad(V2, ((0, NPAD - N), (0, 0)))
    node_pad = _node_mlp(V_pad, sf, c0p, c1p, W1n, b1n, W2n, b2n, NPAD, NW)

    return node_pad[:N][None], edge_emb[None]


# speedup vs baseline: 3.4688x; 1.2878x over previous
import functools

import jax
import jax.numpy as jnp
from jax import lax
from jax.experimental import pallas as pl
from jax.experimental.pallas import tpu as pltpu
from jax.experimental.pallas import tpu_sc as plsc

F32 = jnp.float32


def _cdiv(a, b):
    return (a + b - 1) // b


def _build_gather(N, D, En, NPAD, NW, mesh):
    CG = 512
    G = En // CG
    KMAX = _cdiv(G, NW)
    NC = 2

    @functools.partial(
        pl.kernel,
        out_type=(
            jax.ShapeDtypeStruct((En, D), F32),
            jax.ShapeDtypeStruct((En, D), F32),
            jax.ShapeDtypeStruct((NW, NPAD), F32),
            jax.ShapeDtypeStruct((NW, NPAD), F32),
        ),
        mesh=mesh,
        scratch_types=[
            pltpu.VMEM((CG,), jnp.int32),
            pltpu.VMEM((CG, D), F32),
            pltpu.VMEM((NPAD,), F32),
            pltpu.VMEM((NPAD,), F32),
            pltpu.SemaphoreType.DMA,
        ],
        compiler_params=pltpu.CompilerParams(needs_layout_passes=False),
    )
    def gather_k(v_hbm, i0_hbm, i1_hbm, zc_hbm,
                 s_out, r_out, c0_out, c1_out,
                 idx_v, rows_v, c0_v, c1_v, gsem):
        cid = lax.axis_index("c")
        sid = lax.axis_index("s")
        wid = sid * NC + cid
        ones16 = jnp.full((16,), 1.0, dtype=F32)

        pltpu.sync_copy(zc_hbm, c0_v)
        pltpu.sync_copy(zc_hbm, c1_v)

        def one_array(i_hbm, out_hbm, c_v):
            def body(k, carry):
                g = k * NW + wid

                @pl.when(g < G)
                def _():
                    base = pl.multiple_of(g * CG, CG)
                    pltpu.sync_copy(i_hbm.at[pl.ds(base, CG)], idx_v)
                    descs = [
                        pltpu.async_copy(
                            v_hbm.at[idx_v.at[pl.ds(j * 128, 128)]],
                            rows_v.at[pl.ds(j * 128, 128)],
                            gsem,
                        )
                        for j in range(CG // 128)
                    ]
                    for j in range(CG // 16):
                        plsc.addupdate_scatter(
                            c_v, [idx_v[pl.ds(j * 16, 16)]], ones16)
                    for d in descs:
                        d.wait()
                    pltpu.sync_copy(rows_v, out_hbm.at[pl.ds(base, CG)])

                return carry

            lax.fori_loop(0, KMAX, body, 0)

        one_array(i0_hbm, s_out, c0_v)
        one_array(i1_hbm, r_out, c1_v)

        pltpu.sync_copy(c0_v, c0_out.at[wid])
        pltpu.sync_copy(c1_v, c1_out.at[wid])

    return gather_k


def _build_scatter(NPAD, En, NW, mesh):
    H = 64
    G = En // 128
    NS = 16
    KMAX = _cdiv(G, NS)
    RPT = NPAD // 16

    @functools.partial(
        pl.kernel,
        out_type=jax.ShapeDtypeStruct((2 * NPAD, H), F32),
        mesh=mesh,
        scratch_types=[
            pltpu.VMEM_SHARED((NPAD, H), F32),
            pltpu.VMEM((1, 128), jnp.int32),
            pltpu.VMEM((128, H), F32),
            pltpu.VMEM((RPT, H), F32),
        ],
        compiler_params=pltpu.CompilerParams(use_tc_tiling_on_sc=False),
    )
    def scatter_k(ee_hbm, idx_hbm, z64_hbm,
                  s_out,
                  s_sh, idx_v, e_v, wr_v):
        cid = lax.axis_index("c")
        sid = lax.axis_index("s")

        pltpu.sync_copy(z64_hbm, s_sh.at[pl.ds(sid * RPT, RPT)])
        plsc.subcore_barrier()

        def body(k, carry):
            g = k * NS + sid

            @pl.when(g < G)
            def _():
                pltpu.sync_copy(idx_hbm.at[pl.ds(2 * g + cid, 1)], idx_v)
                pltpu.sync_copy(
                    ee_hbm.at[pl.ds(128 * g, 128), pl.ds(H * cid, H)], e_v)
                pltpu.sync_copy(e_v, s_sh.at[idx_v.at[0]], add=True)

            return carry

        lax.fori_loop(0, KMAX, body, 0)
        plsc.subcore_barrier()

        off = cid * NPAD + sid * RPT
        pltpu.sync_copy(s_sh.at[pl.ds(sid * RPT, RPT)], wr_v)
        pltpu.sync_copy(wr_v, s_out.at[pl.ds(off, RPT)])

    return scatter_k


def _edge_mlp(senders, receivers, E2, W1e, b1e, W2e, b2e):
    En, D = E2.shape
    BE = 512
    grid = En // BE

    def body(s_ref, r_ref, e_ref, w1_ref, b1_ref, w2_ref, b2_ref, o_ref):
        x = jnp.concatenate([s_ref[...], r_ref[...], e_ref[...]], axis=1)
        h = jnp.dot(x, w1_ref[...], preferred_element_type=F32) + b1_ref[...]
        h = h * (1.0 / (1.0 + jnp.exp(-h)))
        o_ref[...] = (
            jnp.dot(h, w2_ref[...], preferred_element_type=F32) + b2_ref[...]
        )

    return pl.pallas_call(
        body,
        grid=(grid,),
        in_specs=[
            pl.BlockSpec((BE, D), lambda i: (i, 0)),
            pl.BlockSpec((BE, D), lambda i: (i, 0)),
            pl.BlockSpec((BE, D), lambda i: (i, 0)),
            pl.BlockSpec(W1e.shape, lambda i: (0, 0)),
            pl.BlockSpec((1, 128), lambda i: (0, 0)),
            pl.BlockSpec(W2e.shape, lambda i: (0, 0)),
            pl.BlockSpec((1, D), lambda i: (0, 0)),
        ],
        out_specs=pl.BlockSpec((BE, D), lambda i: (i, 0)),
        out_shape=jax.ShapeDtypeStruct((En, D), F32),
    )(senders, receivers, E2, W1e, b1e.reshape(1, -1), W2e, b2e.reshape(1, -1))


def _node_mlp(V_pad, sf, c0p, c1p, W1n, b1n, W2n, b2n, NPAD, NW):
    D = V_pad.shape[1]
    H = 64
    BN = 1024
    grid = NPAD // BN
    sr = sf.reshape(2, NPAD, H)

    def body(v_ref, s_ref, c0_ref, c1_ref,
             w1_ref, b1_ref, w2_ref, b2_ref, o_ref):
        s0 = s_ref[0]
        s1 = s_ref[1]
        c0 = jnp.sum(c0_ref[...], axis=0)[:, None]
        c1 = jnp.sum(c1_ref[...], axis=0)[:, None]
        m0 = s0 / jnp.maximum(c0, 1.0)
        m1 = s1 / jnp.maximum(c1, 1.0)
        x = jnp.concatenate([v_ref[...], m0, m1], axis=1)
        h = jnp.dot(x, w1_ref[...], preferred_element_type=F32) + b1_ref[...]
        h = h * (1.0 / (1.0 + jnp.exp(-h)))
        o_ref[...] = (
            jnp.dot(h, w2_ref[...], preferred_element_type=F32) + b2_ref[...]
        )

    return pl.pallas_call(
        body,
        grid=(grid,),
        in_specs=[
            pl.BlockSpec((BN, D), lambda i: (i, 0)),
            pl.BlockSpec((2, BN, H), lambda i: (0, i, 0)),
            pl.BlockSpec((NW, BN), lambda i: (0, i)),
            pl.BlockSpec((NW, BN), lambda i: (0, i)),
            pl.BlockSpec(W1n.shape, lambda i: (0, 0)),
            pl.BlockSpec((1, 128), lambda i: (0, 0)),
            pl.BlockSpec(W2n.shape, lambda i: (0, 0)),
            pl.BlockSpec((1, D), lambda i: (0, 0)),
        ],
        out_specs=pl.BlockSpec((BN, D), lambda i: (i, 0)),
        out_shape=jax.ShapeDtypeStruct((NPAD, D), F32),
    )(V_pad, sr, c0p, c1p,
      W1n, b1n.reshape(1, -1), W2n, b2n.reshape(1, -1))


def kernel(V, E, edges, W1e, b1e, W2e, b2e, W1n, b1n, W2n, b2n):
    _, N, D = V.shape
    En = E.shape[1]
    NPAD = 10240
    info = plsc.get_sparse_core_info()
    NW = info.num_cores * info.num_subcores
    mesh = plsc.VectorSubcoreMesh(core_axis_name="c", subcore_axis_name="s")

    V2 = V[0]
    E2 = E[0]
    idx0 = edges[0, :, 0]
    idx1 = edges[0, :, 1]
    idx_comb = jnp.stack(
        [idx0.reshape(-1, 128), idx1.reshape(-1, 128)], axis=1
    ).reshape(-1, 128)

    RPT = NPAD // 16
    zc = jnp.zeros((NPAD,), dtype=F32)
    senders, receivers, c0p, c1p = _build_gather(N, D, En, NPAD, NW, mesh)(
        V2, idx0, idx1, zc)

    edge_emb = _edge_mlp(senders, receivers, E2, W1e, b1e, W2e, b2e)

    z64 = jnp.zeros((RPT, 64), dtype=F32)
    sf = _build_scatter(NPAD, En, NW, mesh)(edge_emb, idx_comb, z64)

    V_pad = jnp.pad(V2, ((0, NPAD - N), (0, 0)))
    node_pad = _node_mlp(V_pad, sf, c0p, c1p, W1n, b1n, W2n, b2n, NPAD, NW)

    return node_pad[:N][None], edge_emb[None]
